# baseline (speedup 1.0000x reference)
import jax
import jax.numpy as jnp
from jax import lax
from jax.experimental import pallas as pl
from jax.experimental.pallas import tpu as pltpu
from jax.experimental.pallas import tpu_sc as plsc

N_NODES = 10000
N_PAD = 10240
N_EDGES = 320000
D = 128

NC = 2
NS = 16
NW = NC * NS
EW = N_EDGES // NW
CHUNK = 40
NBUF = 4
NCHUNK = EW // CHUNK
NQUAD = NCHUNK // NBUF
ROWS = N_PAD // NS
WR = CHUNK


def _hist_update(hist_v, idst, p, j, one16, tailmask):
    for g16 in range(2):
        idx = idst[p, j, pl.ds(g16 * 16, 16)]
        plsc.addupdate_scatter(hist_v, [idx], one16)
    idx = idst[p, j, pl.ds(CHUNK - 16, 16)]
    plsc.addupdate_scatter(hist_v, [idx], one16, mask=tailmask)


def _sc_body(feat_hbm, src_hbm, dst_hbm, accp_hbm, histp_hbm,
             isrc, idst, msgs, hist_v,
             gsem0, gsem1, gsem2, gsem3,
             ssem0, ssem1, ssem2, ssem3, isem0, isem1, acc_sh):
    c = lax.axis_index("c")
    s = lax.axis_index("s")
    wid = c * NS + s
    ebase = wid * EW

    zero16 = jnp.zeros((16,), jnp.float32)
    one16 = jnp.ones((16,), jnp.float32)
    tailmask = lax.iota(jnp.int32, 16) >= (2 * 16 - (CHUNK - 16))
    gsems = [gsem0, gsem1, gsem2, gsem3]
    ssems = [ssem0, ssem1, ssem2, ssem3]
    isems = [isem0, isem1]

    for i in range(WR):
        for j in range(D // 16):
            msgs[0, i, pl.ds(j * 16, 16)] = zero16

    base = s * ROWS
    for i in range(ROWS // WR):
        pltpu.sync_copy(msgs.at[0], acc_sh.at[pl.ds(base + i * WR, WR)])

    def hzbody(i, carry):
        hist_v[pl.ds(i * 16, 16)] = zero16
        return carry

    lax.fori_loop(0, N_PAD // 16, hzbody, 0)
    plsc.subcore_barrier()

    def issue_idx(quad, p):
        off = ebase + quad * NBUF * CHUNK
        for j in range(NBUF):
            pltpu.async_copy(src_hbm.at[pl.ds(off + j * CHUNK, CHUNK)],
                             isrc.at[p, j], isems[p])
            pltpu.async_copy(dst_hbm.at[pl.ds(off + j * CHUNK, CHUNK)],
                             idst.at[p, j], isems[p])

    def drain_idx(p):
        for j in range(NBUF):
            pltpu.make_async_copy(src_hbm.at[pl.ds(0, CHUNK)],
                                  isrc.at[p, j], isems[p]).wait()
            pltpu.make_async_copy(dst_hbm.at[pl.ds(0, CHUNK)],
                                  idst.at[p, j], isems[p]).wait()

    issue_idx(0, 0)

    def ebody(g, carry):
        p = lax.rem(g, 2)

        @pl.when(p == 0)
        def _even():
            drain_idx(0)

        @pl.when(p == 1)
        def _odd():
            drain_idx(1)

        @pl.when(g + 1 < NQUAD)
        def _prefetch():
            @pl.when(p == 0)
            def _():
                issue_idx(g + 1, 1)

            @pl.when(p == 1)
            def _():
                issue_idx(g + 1, 0)

        for p_const in range(2):
            @pl.when(p == p_const)
            def _run(p_const=p_const):
                gds = [pltpu.async_copy(
                    feat_hbm.at[isrc.at[p_const, b]], msgs.at[b], gsems[b])
                    for b in range(NBUF)]
                sds = []
                for b in range(NBUF):
                    gds[b].wait()
                    sds.append(pltpu.async_copy(
                        msgs.at[b], acc_sh.at[idst.at[p_const, b]],
                        ssems[b], add=True))
                    _hist_update(hist_v, idst, p_const, b, one16, tailmask)
                for b in range(NBUF):
                    sds[b].wait()
        return carry

    lax.fori_loop(0, NQUAD, ebody, 0)

    for t in range(NQUAD * NBUF, NCHUNK):
        toff = ebase + t * CHUNK
        pltpu.sync_copy(src_hbm.at[pl.ds(toff, CHUNK)], isrc.at[0, 0])
        pltpu.sync_copy(dst_hbm.at[pl.ds(toff, CHUNK)], idst.at[0, 0])
        pltpu.async_copy(feat_hbm.at[isrc.at[0, 0]], msgs.at[0],
                         gsem0).wait()
        pltpu.sync_copy(msgs.at[0], acc_sh.at[idst.at[0, 0]], add=True)
        _hist_update(hist_v, idst, 0, 0, one16, tailmask)
    plsc.subcore_barrier()

    pltpu.sync_copy(hist_v, histp_hbm.at[wid])

    def wbody(i, carry):
        off = base + i * WR
        pltpu.sync_copy(acc_sh.at[pl.ds(off, WR)], msgs.at[0])
        pltpu.sync_copy(msgs.at[0], accp_hbm.at[c, pl.ds(off, WR)])
        return carry

    lax.fori_loop(0, ROWS // WR, wbody, 0)


@jax.jit
def _sc_aggregate(feature, src, dst):
    mesh = plsc.VectorSubcoreMesh(core_axis_name="c", subcore_axis_name="s",
                                  num_cores=NC, num_subcores=NS)
    return pl.kernel(
        _sc_body,
        out_type=(
            jax.ShapeDtypeStruct((NC, N_PAD, D), jnp.float32),
            jax.ShapeDtypeStruct((NW, N_PAD), jnp.float32),
        ),
        mesh=mesh,
        compiler_params=pltpu.CompilerParams(needs_layout_passes=False),
        scratch_types=[
            pltpu.VMEM((2, NBUF, CHUNK), jnp.int32),
            pltpu.VMEM((2, NBUF, CHUNK), jnp.int32),
            pltpu.VMEM((NBUF, CHUNK, D), jnp.float32),
            pltpu.VMEM((N_PAD,), jnp.float32),
            pltpu.SemaphoreType.DMA,
            pltpu.SemaphoreType.DMA,
            pltpu.SemaphoreType.DMA,
            pltpu.SemaphoreType.DMA,
            pltpu.SemaphoreType.DMA,
            pltpu.SemaphoreType.DMA,
            pltpu.SemaphoreType.DMA,
            pltpu.SemaphoreType.DMA,
            pltpu.SemaphoreType.DMA,
            pltpu.SemaphoreType.DMA,
            pltpu.VMEM_SHARED((N_PAD, D), jnp.float32),
        ],
    )(feature, src, dst)


def _tc_body(p0_ref, p1_ref, h_ref, ones_ref, w_ref, b_ref, out_ref):
    deg = lax.dot_general(
        h_ref[...], ones_ref[...], (((0,), (0,)), ((), ())),
        preferred_element_type=jnp.float32)
    h = (p0_ref[...] + p1_ref[...]) / jnp.maximum(deg, 1.0)
    out_ref[...] = lax.dot_general(
        h, w_ref[...], (((1,), (1,)), ((), ())),
        preferred_element_type=jnp.float32) + b_ref[...]


TC_BLK = 2048


@jax.jit
def _tc_finish(accp, histp, W, b):
    return pl.pallas_call(
        _tc_body,
        grid=(N_PAD // TC_BLK,),
        in_specs=[
            pl.BlockSpec((TC_BLK, D), lambda i: (i, 0)),
            pl.BlockSpec((TC_BLK, D), lambda i: (i, 0)),
            pl.BlockSpec((NW, TC_BLK), lambda i: (0, i)),
            pl.BlockSpec((NW, 1), lambda i: (0, 0)),
            pl.BlockSpec((D, D), lambda i: (0, 0)),
            pl.BlockSpec((1, D), lambda i: (0, 0)),
        ],
        out_specs=pl.BlockSpec((TC_BLK, D), lambda i: (i, 0)),
        out_shape=jax.ShapeDtypeStruct((N_PAD, D), jnp.float32),
    )(accp[0], accp[1], histp, jnp.ones((NW, 1), jnp.float32), W,
      b.reshape(1, D))


def kernel(feature, edge_index, W, b):
    ei = edge_index.astype(jnp.int32)
    accp, histp = _sc_aggregate(feature, ei[0], ei[1])
    return _tc_finish(accp, histp, W, b)[:N_NODES]


# speedup vs baseline: 1.0300x; 1.0300x over previous
import jax
import jax.numpy as jnp
from jax import lax
from jax.experimental import pallas as pl
from jax.experimental.pallas import tpu as pltpu
from jax.experimental.pallas import tpu_sc as plsc

N_NODES = 10000
N_PAD = 10240
N_EDGES = 320000
D = 128

NC = 2
NS = 16
NW = NC * NS
EW = N_EDGES // NW
CHUNK = 80
NCHUNK = EW // CHUNK
NPAIR = NCHUNK // 2
ROWS = N_PAD // NS
WR = CHUNK


def _hist_update(hist_v, idst, p, j, one16):
    for g16 in range(CHUNK // 16):
        idx = idst[p, j, pl.ds(g16 * 16, 16)]
        plsc.addupdate_scatter(hist_v, [idx], one16)


def _sc_body(feat_hbm, src_hbm, dst_hbm, accp_hbm, histp_hbm,
             isrc, idst, msgs, hist_v,
             gsem0, gsem1, ssem0, ssem1, isem0, isem1, acc_sh):
    c = lax.axis_index("c")
    s = lax.axis_index("s")
    wid = c * NS + s
    ebase = wid * EW

    zero16 = jnp.zeros((16,), jnp.float32)
    one16 = jnp.ones((16,), jnp.float32)
    gsems = [gsem0, gsem1]
    ssems = [ssem0, ssem1]
    isems = [isem0, isem1]

    for i in range(WR):
        for j in range(D // 16):
            msgs[0, i, pl.ds(j * 16, 16)] = zero16

    base = s * ROWS
    for i in range(ROWS // WR):
        pltpu.sync_copy(msgs.at[0], acc_sh.at[pl.ds(base + i * WR, WR)])

    def hzbody(i, carry):
        hist_v[pl.ds(i * 16, 16)] = zero16
        return carry

    lax.fori_loop(0, N_PAD // 16, hzbody, 0)
    plsc.subcore_barrier()

    def issue_idx(pair, p):
        off = ebase + pair * 2 * CHUNK
        for j in range(2):
            pltpu.async_copy(src_hbm.at[pl.ds(off + j * CHUNK, CHUNK)],
                             isrc.at[p, j], isems[p])
            pltpu.async_copy(dst_hbm.at[pl.ds(off + j * CHUNK, CHUNK)],
                             idst.at[p, j], isems[p])

    def drain_idx(p):
        for j in range(2):
            pltpu.make_async_copy(src_hbm.at[pl.ds(0, CHUNK)],
                                  isrc.at[p, j], isems[p]).wait()
            pltpu.make_async_copy(dst_hbm.at[pl.ds(0, CHUNK)],
                                  idst.at[p, j], isems[p]).wait()

    issue_idx(0, 0)

    def ebody(g, carry):
        p = lax.rem(g, 2)

        @pl.when(p == 0)
        def _even():
            drain_idx(0)

        @pl.when(p == 1)
        def _odd():
            drain_idx(1)

        @pl.when(g + 1 < NPAIR)
        def _prefetch():
            @pl.when(p == 0)
            def _():
                issue_idx(g + 1, 1)

            @pl.when(p == 1)
            def _():
                issue_idx(g + 1, 0)

        for p_const in range(2):
            @pl.when(p == p_const)
            def _run(p_const=p_const):
                gd0 = pltpu.async_copy(
                    feat_hbm.at[isrc.at[p_const, 0]], msgs.at[0], gsems[0])
                gd1 = pltpu.async_copy(
                    feat_hbm.at[isrc.at[p_const, 1]], msgs.at[1], gsems[1])
                gd0.wait()
                sd0 = pltpu.async_copy(
                    msgs.at[0], acc_sh.at[idst.at[p_const, 0]], ssems[0],
                    add=True)
                _hist_update(hist_v, idst, p_const, 0, one16)
                gd1.wait()
                sd1 = pltpu.async_copy(
                    msgs.at[1], acc_sh.at[idst.at[p_const, 1]], ssems[1],
                    add=True)
                _hist_update(hist_v, idst, p_const, 1, one16)
                sd0.wait()
                sd1.wait()
        return carry

    lax.fori_loop(0, NPAIR, ebody, 0)

    toff = ebase + (NCHUNK - 1) * CHUNK
    pltpu.sync_copy(src_hbm.at[pl.ds(toff, CHUNK)], isrc.at[0, 0])
    pltpu.sync_copy(dst_hbm.at[pl.ds(toff, CHUNK)], idst.at[0, 0])
    pltpu.async_copy(feat_hbm.at[isrc.at[0, 0]], msgs.at[0], gsem0).wait()
    pltpu.sync_copy(msgs.at[0], acc_sh.at[idst.at[0, 0]], add=True)
    _hist_update(hist_v, idst, 0, 0, one16)
    plsc.subcore_barrier()

    pltpu.sync_copy(hist_v, histp_hbm.at[wid])

    def wbody(i, carry):
        off = base + i * WR
        pltpu.sync_copy(acc_sh.at[pl.ds(off, WR)], msgs.at[0])
        pltpu.sync_copy(msgs.at[0], accp_hbm.at[c, pl.ds(off, WR)])
        return carry

    lax.fori_loop(0, ROWS // WR, wbody, 0)


@jax.jit
def _sc_aggregate(feature, src, dst):
    mesh = plsc.VectorSubcoreMesh(core_axis_name="c", subcore_axis_name="s",
                                  num_cores=NC, num_subcores=NS)
    return pl.kernel(
        _sc_body,
        out_type=(
            jax.ShapeDtypeStruct((NC, N_PAD, D), jnp.float32),
            jax.ShapeDtypeStruct((NW, N_PAD), jnp.float32),
        ),
        mesh=mesh,
        compiler_params=pltpu.CompilerParams(needs_layout_passes=False),
        scratch_types=[
            pltpu.VMEM((2, 2, CHUNK), jnp.int32),
            pltpu.VMEM((2, 2, CHUNK), jnp.int32),
            pltpu.VMEM((2, CHUNK, D), jnp.float32),
            pltpu.VMEM((N_PAD,), jnp.float32),
            pltpu.SemaphoreType.DMA,
            pltpu.SemaphoreType.DMA,
            pltpu.SemaphoreType.DMA,
            pltpu.SemaphoreType.DMA,
            pltpu.SemaphoreType.DMA,
            pltpu.SemaphoreType.DMA,
            pltpu.VMEM_SHARED((N_PAD, D), jnp.float32),
        ],
    )(feature, src, dst)


def _tc_body(p0_ref, p1_ref, h_ref, ones_ref, w_ref, b_ref, out_ref):
    deg = lax.dot_general(
        h_ref[...], ones_ref[...], (((0,), (0,)), ((), ())),
        preferred_element_type=jnp.float32)
    h = (p0_ref[...] + p1_ref[...]) / jnp.maximum(deg, 1.0)
    out_ref[...] = lax.dot_general(
        h, w_ref[...], (((1,), (1,)), ((), ())),
        preferred_element_type=jnp.float32) + b_ref[...]


TC_BLK = 2048


@jax.jit
def _tc_finish(accp, histp, W, b):
    return pl.pallas_call(
        _tc_body,
        grid=(N_PAD // TC_BLK,),
        in_specs=[
            pl.BlockSpec((TC_BLK, D), lambda i: (i, 0)),
            pl.BlockSpec((TC_BLK, D), lambda i: (i, 0)),
            pl.BlockSpec((NW, TC_BLK), lambda i: (0, i)),
            pl.BlockSpec((NW, 1), lambda i: (0, 0)),
            pl.BlockSpec((D, D), lambda i: (0, 0)),
            pl.BlockSpec((1, D), lambda i: (0, 0)),
        ],
        out_specs=pl.BlockSpec((TC_BLK, D), lambda i: (i, 0)),
        out_shape=jax.ShapeDtypeStruct((N_NODES, D), jnp.float32),
    )(accp[0], accp[1], histp, jnp.ones((NW, 1), jnp.float32), W,
      b.reshape(1, D))


def kernel(feature, edge_index, W, b):
    ei = edge_index.astype(jnp.int32)
    accp, histp = _sc_aggregate(feature, ei[0], ei[1])
    return _tc_finish(accp, histp, W, b)
